# baseline (speedup 1.0000x reference)
import jax
import jax.numpy as jnp
from jax.experimental import pallas as pl

_N = 10000
_E = 320000
_DN = 128
_DE = 16
_DO = 128

_NBLK = 1000
_EBLK = 8000


def _node_proj_kernel(fn_ref, wn_ref, ws_ref, wd_ref, z_ref, s_ref, d_ref):
    z = jnp.dot(fn_ref[...], wn_ref[...].T, preferred_element_type=jnp.float32)
    z_ref[...] = z
    s_ref[...] = jnp.dot(z, ws_ref[...], preferred_element_type=jnp.float32)
    d_ref[...] = jnp.dot(z, wd_ref[...], preferred_element_type=jnp.float32)


def _edge_proj_kernel(fe_ref, we_ref, wa_ref, ze_ref, ee_ref):
    ze = jnp.dot(fe_ref[...], we_ref[...].T, preferred_element_type=jnp.float32)
    ze_ref[...] = ze
    ee_ref[...] = jnp.dot(ze, wa_ref[...], preferred_element_type=jnp.float32)


def _score_kernel(ssrc_ref, ddst_ref, ee_ref, e_ref):
    a = ssrc_ref[...] + ddst_ref[...] + ee_ref[...]
    e_ref[...] = jnp.where(a > 0, a, 0.2 * a)


def _exp_kernel(e_ref, emaxd_ref, ex_ref):
    ex_ref[...] = jnp.exp(e_ref[...] - emaxd_ref[...])


def _weight_kernel(ex_ref, dend_ref, zsrc_ref, out_ref):
    out_ref[...] = (ex_ref[...] / dend_ref[...]) * zsrc_ref[...]


def kernel(feats_node, feats_edge, edge_index, W_node, W_edge, W_attn):
    src = edge_index[0].astype(jnp.int32)
    dst = edge_index[1].astype(jnp.int32)
    w = W_attn[0]
    ws = w[:_DO].reshape(_DO, 1)
    wd = w[_DO:2 * _DO].reshape(_DO, 1)
    we = w[2 * _DO:].reshape(_DE, 1)

    z, s, d = pl.pallas_call(
        _node_proj_kernel,
        grid=(_N // _NBLK,),
        in_specs=[
            pl.BlockSpec((_NBLK, _DN), lambda i: (i, 0)),
            pl.BlockSpec((_DO, _DN), lambda i: (0, 0)),
            pl.BlockSpec((_DO, 1), lambda i: (0, 0)),
            pl.BlockSpec((_DO, 1), lambda i: (0, 0)),
        ],
        out_specs=[
            pl.BlockSpec((_NBLK, _DO), lambda i: (i, 0)),
            pl.BlockSpec((_NBLK, 1), lambda i: (i, 0)),
            pl.BlockSpec((_NBLK, 1), lambda i: (i, 0)),
        ],
        out_shape=[
            jax.ShapeDtypeStruct((_N, _DO), jnp.float32),
            jax.ShapeDtypeStruct((_N, 1), jnp.float32),
            jax.ShapeDtypeStruct((_N, 1), jnp.float32),
        ],
    )(feats_node, W_node, ws, wd)

    ze, ee = pl.pallas_call(
        _edge_proj_kernel,
        grid=(_E // _EBLK,),
        in_specs=[
            pl.BlockSpec((_EBLK, _DE), lambda i: (i, 0)),
            pl.BlockSpec((_DE, _DE), lambda i: (0, 0)),
            pl.BlockSpec((_DE, 1), lambda i: (0, 0)),
        ],
        out_specs=[
            pl.BlockSpec((_EBLK, _DE), lambda i: (i, 0)),
            pl.BlockSpec((_EBLK, 1), lambda i: (i, 0)),
        ],
        out_shape=[
            jax.ShapeDtypeStruct((_E, _DE), jnp.float32),
            jax.ShapeDtypeStruct((_E, 1), jnp.float32),
        ],
    )(feats_edge, W_edge, we)

    s_flat = s[:, 0]
    d_flat = d[:, 0]

    rows = _E // 128
    ssrc = jnp.take(s_flat, src).reshape(rows, 128)
    ddst = jnp.take(d_flat, dst).reshape(rows, 128)
    ee2 = ee.reshape(rows, 128)
    e = pl.pallas_call(
        _score_kernel,
        grid=(rows // 2500,),
        in_specs=[pl.BlockSpec((2500, 128), lambda i: (i, 0))] * 3,
        out_specs=pl.BlockSpec((2500, 128), lambda i: (i, 0)),
        out_shape=jax.ShapeDtypeStruct((rows, 128), jnp.float32),
    )(ssrc, ddst, ee2)
    e_flat = e.reshape(_E)

    emax = jax.ops.segment_max(e_flat, dst, num_segments=_N)
    emax = jnp.where(jnp.isfinite(emax), emax, 0.0)
    emaxd = jnp.take(emax, dst).reshape(rows, 128)
    ex = pl.pallas_call(
        _exp_kernel,
        grid=(rows // 2500,),
        in_specs=[pl.BlockSpec((2500, 128), lambda i: (i, 0))] * 2,
        out_specs=pl.BlockSpec((2500, 128), lambda i: (i, 0)),
        out_shape=jax.ShapeDtypeStruct((rows, 128), jnp.float32),
    )(e, emaxd)
    ex_flat = ex.reshape(_E)
    denom = jax.ops.segment_sum(ex_flat, dst, num_segments=_N)

    zsrc = jnp.take(z, src, axis=0)
    exc = ex_flat.reshape(_E, 1)
    dend = jnp.take(denom, dst).reshape(_E, 1)
    weighted = pl.pallas_call(
        _weight_kernel,
        grid=(_E // _EBLK,),
        in_specs=[
            pl.BlockSpec((_EBLK, 1), lambda i: (i, 0)),
            pl.BlockSpec((_EBLK, 1), lambda i: (i, 0)),
            pl.BlockSpec((_EBLK, _DO), lambda i: (i, 0)),
        ],
        out_specs=pl.BlockSpec((_EBLK, _DO), lambda i: (i, 0)),
        out_shape=jax.ShapeDtypeStruct((_E, _DO), jnp.float32),
    )(exc, dend, zsrc)
    h = jax.ops.segment_sum(weighted, dst, num_segments=_N)
    return (h, ze)


# speedup vs baseline: 1.0395x; 1.0395x over previous
import jax
import jax.numpy as jnp
from jax.experimental import pallas as pl

_N = 10000
_E = 320000
_DN = 128
_DE = 16
_DO = 128

_NBLK = 1000
_EBLK = 8000


def _node_proj_kernel(fn_ref, wn_ref, ws_ref, wd_ref, z_ref, s_ref, d_ref):
    z = jnp.dot(fn_ref[...], wn_ref[...].T, preferred_element_type=jnp.float32)
    z_ref[...] = z
    s_ref[...] = jnp.dot(z, ws_ref[...], preferred_element_type=jnp.float32)
    d_ref[...] = jnp.dot(z, wd_ref[...], preferred_element_type=jnp.float32)


def _edge_proj_kernel(fe_ref, we_ref, wa_ref, ze_ref, ee_ref):
    ze = jnp.dot(fe_ref[...], we_ref[...].T, preferred_element_type=jnp.float32)
    ze_ref[...] = ze
    ee_ref[...] = jnp.dot(ze, wa_ref[...], preferred_element_type=jnp.float32)


def _score_kernel(ssrc_ref, ddst_ref, ee_ref, e_ref):
    a = ssrc_ref[...] + ddst_ref[...] + ee_ref[...]
    e_ref[...] = jnp.where(a > 0, a, 0.2 * a)


def _exp_kernel(e_ref, emaxd_ref, ex_ref):
    ex_ref[...] = jnp.exp(e_ref[...] - emaxd_ref[...])


def _alpha_kernel(ex_ref, dend_ref, out_ref):
    out_ref[...] = ex_ref[...] / dend_ref[...]


def kernel(feats_node, feats_edge, edge_index, W_node, W_edge, W_attn):
    src = edge_index[0].astype(jnp.int32)
    dst = edge_index[1].astype(jnp.int32)
    w = W_attn[0]
    ws = w[:_DO].reshape(_DO, 1)
    wd = w[_DO:2 * _DO].reshape(_DO, 1)
    we = w[2 * _DO:].reshape(_DE, 1)

    z, s, d = pl.pallas_call(
        _node_proj_kernel,
        grid=(_N // _NBLK,),
        in_specs=[
            pl.BlockSpec((_NBLK, _DN), lambda i: (i, 0)),
            pl.BlockSpec((_DO, _DN), lambda i: (0, 0)),
            pl.BlockSpec((_DO, 1), lambda i: (0, 0)),
            pl.BlockSpec((_DO, 1), lambda i: (0, 0)),
        ],
        out_specs=[
            pl.BlockSpec((_NBLK, _DO), lambda i: (i, 0)),
            pl.BlockSpec((_NBLK, 1), lambda i: (i, 0)),
            pl.BlockSpec((_NBLK, 1), lambda i: (i, 0)),
        ],
        out_shape=[
            jax.ShapeDtypeStruct((_N, _DO), jnp.float32),
            jax.ShapeDtypeStruct((_N, 1), jnp.float32),
            jax.ShapeDtypeStruct((_N, 1), jnp.float32),
        ],
    )(feats_node, W_node, ws, wd)

    ze, ee = pl.pallas_call(
        _edge_proj_kernel,
        grid=(_E // _EBLK,),
        in_specs=[
            pl.BlockSpec((_EBLK, _DE), lambda i: (i, 0)),
            pl.BlockSpec((_DE, _DE), lambda i: (0, 0)),
            pl.BlockSpec((_DE, 1), lambda i: (0, 0)),
        ],
        out_specs=[
            pl.BlockSpec((_EBLK, _DE), lambda i: (i, 0)),
            pl.BlockSpec((_EBLK, 1), lambda i: (i, 0)),
        ],
        out_shape=[
            jax.ShapeDtypeStruct((_E, _DE), jnp.float32),
            jax.ShapeDtypeStruct((_E, 1), jnp.float32),
        ],
    )(feats_edge, W_edge, we)

    s_flat = s[:, 0]
    d_flat = d[:, 0]

    rows = _E // 128
    ssrc = jnp.take(s_flat, src).reshape(rows, 128)
    ddst = jnp.take(d_flat, dst).reshape(rows, 128)
    ee2 = ee.reshape(rows, 128)
    e = pl.pallas_call(
        _score_kernel,
        grid=(rows // 2500,),
        in_specs=[pl.BlockSpec((2500, 128), lambda i: (i, 0))] * 3,
        out_specs=pl.BlockSpec((2500, 128), lambda i: (i, 0)),
        out_shape=jax.ShapeDtypeStruct((rows, 128), jnp.float32),
    )(ssrc, ddst, ee2)
    e_flat = e.reshape(_E)

    emax = jax.ops.segment_max(e_flat, dst, num_segments=_N)
    emax = jnp.where(jnp.isfinite(emax), emax, 0.0)
    emaxd = jnp.take(emax, dst).reshape(rows, 128)
    ex = pl.pallas_call(
        _exp_kernel,
        grid=(rows // 2500,),
        in_specs=[pl.BlockSpec((2500, 128), lambda i: (i, 0))] * 2,
        out_specs=pl.BlockSpec((2500, 128), lambda i: (i, 0)),
        out_shape=jax.ShapeDtypeStruct((rows, 128), jnp.float32),
    )(e, emaxd)
    ex_flat = ex.reshape(_E)
    denom = jax.ops.segment_sum(ex_flat, dst, num_segments=_N)

    dend = jnp.take(denom, dst).reshape(rows, 128)
    alpha = pl.pallas_call(
        _alpha_kernel,
        grid=(rows // 2500,),
        in_specs=[pl.BlockSpec((2500, 128), lambda i: (i, 0))] * 2,
        out_specs=pl.BlockSpec((2500, 128), lambda i: (i, 0)),
        out_shape=jax.ShapeDtypeStruct((rows, 128), jnp.float32),
    )(ex, dend)
    alpha_col = alpha.reshape(_E, 1)
    h = jax.ops.segment_sum(
        alpha_col * jnp.take(z, src, axis=0), dst, num_segments=_N)
    return (h, ze)
